# baseline (speedup 1.0000x reference)
import jax
import jax.numpy as jnp
from jax import lax
from jax.experimental import pallas as pl
from jax.experimental.pallas import tpu as pltpu
from jax.experimental.pallas import tpu_sc as plsc

B, N, M = 4096, 2048, 2048
W = N + M
NV = W // 16
NVH = NV // 2
NC, NS = 2, 16
NWORK = NC * NS
RPW = B // NWORK
NDIG = 256
NPASS = 4
HB = W
UNROLL = 8


def _sc_body(x_hbm, y_hbm, xp_hbm, yp_hbm, out_hbm,
             key_a, val_a, key_b, val_b, key_s, val_s, hist, off, loss_buf,
             dma_sem):
    wid = lax.axis_index("s") * NC + lax.axis_index("c")
    base = wid * RPW
    lanes = lax.iota(jnp.int32, 16)
    ones_i = jnp.full((16,), 1, jnp.int32)
    zeros_i = jnp.zeros((16,), jnp.int32)
    zeros_f = jnp.zeros((16,), jnp.float32)
    ones_f = jnp.ones((16,), jnp.float32)

    def zero_body(i, _):
        hist[pl.ds(i * 16, 16)] = zeros_i
        return 0

    lax.fori_loop(0, 2 * NV, zero_body, 0, unroll=UNROLL)

    def issue_prefetch(row):
        pltpu.async_copy(xp_hbm.at[row], key_s.at[pl.ds(0, N)], dma_sem)
        pltpu.async_copy(yp_hbm.at[row], key_s.at[pl.ds(N, M)], dma_sem)
        pltpu.async_copy(x_hbm.at[row], val_s.at[pl.ds(0, N)], dma_sem)
        pltpu.async_copy(y_hbm.at[row], val_s.at[pl.ds(N, M)], dma_sem)

    issue_prefetch(base)

    def row_body(r, _):
        row = base + r
        pltpu.make_async_copy(xp_hbm.at[row], key_s.at[pl.ds(0, N)], dma_sem).wait()
        pltpu.make_async_copy(yp_hbm.at[row], key_s.at[pl.ds(N, M)], dma_sem).wait()
        pltpu.make_async_copy(x_hbm.at[row], val_s.at[pl.ds(0, N)], dma_sem).wait()
        pltpu.make_async_copy(y_hbm.at[row], val_s.at[pl.ds(N, M)], dma_sem).wait()

        scale_vecs = [None]

        for p in range(NPASS):
            shift = 8 * p
            if p == 0:
                src_k, src_v = key_s, val_s
            elif p % 2 == 0:
                src_k, src_v = key_b, val_b
            else:
                src_k, src_v = key_a, val_a
            dst_k, dst_v = (key_a, val_a) if p % 2 == 0 else (key_b, val_b)

            def digit(vreg_i, half, shift=shift, src_k=src_k):
                k = src_k[pl.ds((vreg_i + half * NVH) * 16, 16)]
                ki = plsc.bitcast(k, jnp.int32)
                d = lax.shift_right_logical(ki, shift) & (NDIG - 1)
                return k, d

            if p == 0:
                def hist0_body(i, carry):
                    ax, ay = carry
                    _, da = digit(i, 0)
                    plsc.addupdate_scatter(hist, [(lanes << 8) | da], ones_i)
                    _, db = digit(i, 1)
                    plsc.addupdate_scatter(hist, [HB + ((lanes << 8) | db)], ones_i)
                    ax = ax + src_v[pl.ds(i * 16, 16)]
                    ay = ay + src_v[pl.ds((i + NVH) * 16, 16)]
                    return ax, ay

                ax, ay = lax.fori_loop(0, NVH, hist0_body, (zeros_f, zeros_f),
                                       unroll=UNROLL)
                inv_sx = ones_f / (ones_f * jnp.sum(ax))
                neg_inv_sy = (-ones_f) / (ones_f * jnp.sum(ay))
                scale_vecs[0] = (inv_sx, neg_inv_sy)
            else:
                def hist_body(i, _, digit=digit):
                    _, da = digit(i, 0)
                    plsc.addupdate_scatter(hist, [(lanes << 8) | da], ones_i)
                    _, db = digit(i, 1)
                    plsc.addupdate_scatter(hist, [HB + ((lanes << 8) | db)], ones_i)
                    return 0

                lax.fori_loop(0, NVH, hist_body, 0, unroll=UNROLL)

            ngroups = (NDIG // 16) if p < NPASS - 1 else 4

            def off_group(j, carry):
                acc = zeros_i
                es, has = [], []
                for l in range(16):
                    sa = pl.ds(l * NDIG + j * 16, 16)
                    sb = pl.ds(HB + l * NDIG + j * 16, 16)
                    ha = hist[sa]
                    hb = hist[sb]
                    hist[sa] = zeros_i
                    hist[sb] = zeros_i
                    es.append(acc)
                    has.append(ha)
                    acc = acc + ha + hb
                incl = plsc.cumsum(acc)
                base_vec = carry + incl - acc
                for l in range(16):
                    oa = base_vec + es[l]
                    off[pl.ds(l * NDIG + j * 16, 16)] = oa
                    off[pl.ds(HB + l * NDIG + j * 16, 16)] = oa + has[l]
                return carry + incl[15]

            lax.fori_loop(0, ngroups, off_group, jnp.int32(0))

            if p == 0:
                inv_sx, neg_inv_sy = scale_vecs[0]
                scales = (inv_sx, neg_inv_sy)
            else:
                scales = None

            def perm_body(i, _, digit=digit, src_v=src_v, dst_k=dst_k,
                          dst_v=dst_v, scales=scales):
                for half in (0, 1):
                    k, d = digit(i, half)
                    v = src_v[pl.ds((i + half * NVH) * 16, 16)]
                    taddr = half * HB + ((lanes << 8) | d)
                    g = plsc.load_gather(off, [taddr])
                    plsc.addupdate_scatter(off, [taddr], ones_i)
                    a = ((g & (NV - 1)) << 4) | lax.shift_right_logical(g, 8)
                    plsc.store_scatter(dst_k, [a], k)
                    if scales is not None:
                        v = v * scales[half]
                    plsc.store_scatter(dst_v, [a], v)
                return 0

            lax.fori_loop(0, NVH, perm_body, 0, unroll=4)

            if p == 0:
                @pl.when(r + 1 < RPW)
                def _():
                    issue_prefetch(row + 1)

        def tot_body(i, acc):
            return acc + val_b[pl.ds(i * 16, 16)]

        t = lax.fori_loop(0, NV, tot_body, zeros_f, unroll=UNROLL)
        lane_base = plsc.cumsum(t) - t

        def loss_body(i, carry):
            k, acc, lsum = carry
            nk = key_b[pl.ds((i + 1) * 16, 16)]
            acc = acc + val_b[pl.ds(i * 16, 16)]
            lsum = lsum + (nk - k) * jnp.abs(lane_base + acc)
            return nk, acc, lsum

        k_end, acc, lsum = lax.fori_loop(
            0, NV - 1, loss_body,
            (key_b[pl.ds(0, 16)], zeros_f, zeros_f), unroll=UNROLL)
        acc = acc + val_b[pl.ds((NV - 1) * 16, 16)]
        d_end = lane_base + acc
        nxt_start = plsc.load_gather(key_b, [jnp.minimum(lanes + 1, 15)])
        seam = jnp.where(lanes < 15, (nxt_start - k_end) * jnp.abs(d_end), 0.0)
        loss = jnp.sum(lsum) + jnp.sum(seam)
        plsc.store_scatter(loss_buf, [ones_i * r], zeros_f + loss,
                           mask=lanes == 0)
        return 0

    lax.fori_loop(0, RPW, row_body, 0)
    pltpu.sync_copy(loss_buf, out_hbm.at[pl.ds(base, RPW)])


@jax.jit
def kernel(x, y, x_pos, y_pos):
    mesh = plsc.VectorSubcoreMesh(core_axis_name="c", subcore_axis_name="s",
                                  num_cores=NC, num_subcores=NS)
    f = pl.kernel(
        _sc_body,
        out_type=jax.ShapeDtypeStruct((B,), jnp.float32),
        mesh=mesh,
        scratch_types=[
            pltpu.VMEM((W,), jnp.float32),
            pltpu.VMEM((W,), jnp.float32),
            pltpu.VMEM((W,), jnp.float32),
            pltpu.VMEM((W,), jnp.float32),
            pltpu.VMEM((W,), jnp.float32),
            pltpu.VMEM((W,), jnp.float32),
            pltpu.VMEM((2 * W,), jnp.int32),
            pltpu.VMEM((2 * W,), jnp.int32),
            pltpu.VMEM((RPW,), jnp.float32),
            pltpu.SemaphoreType.DMA,
        ],
        compiler_params=pltpu.CompilerParams(needs_layout_passes=False),
    )
    return f(x, y, x_pos, y_pos)


# speedup vs baseline: 1.0051x; 1.0051x over previous
import jax
import jax.numpy as jnp
from jax import lax
from jax.experimental import pallas as pl
from jax.experimental.pallas import tpu as pltpu
from jax.experimental.pallas import tpu_sc as plsc

B, N, M = 4096, 2048, 2048
W = N + M
NV = W // 16
NVH = NV // 2
NC, NS = 2, 16
NWORK = NC * NS
RPW = B // NWORK
NPAIR = RPW // 2
NDIG = 256
NPASS = 4
HB = W
UNROLL = 8


def _sc_body(x_hbm, y_hbm, xp_hbm, yp_hbm, out_hbm,
             ka0, va0, kb0, vb0, ks0, vs0, h0, o0,
             ka1, va1, kb1, vb1, ks1, vs1, h1, o1,
             loss_buf, dma_sem):
    wid = lax.axis_index("s") * NC + lax.axis_index("c")
    base = wid * RPW
    lanes = lax.iota(jnp.int32, 16)
    ones_i = jnp.full((16,), 1, jnp.int32)
    zeros_i = jnp.zeros((16,), jnp.int32)
    zeros_f = jnp.zeros((16,), jnp.float32)
    ones_f = jnp.ones((16,), jnp.float32)

    KA, VA = (ka0, ka1), (va0, va1)
    KB, VB = (kb0, kb1), (vb0, vb1)
    KS, VS = (ks0, ks1), (vs0, vs1)
    HI, OF = (h0, h1), (o0, o1)

    def zero_body(i, _):
        h0[pl.ds(i * 16, 16)] = zeros_i
        h1[pl.ds(i * 16, 16)] = zeros_i
        return 0

    lax.fori_loop(0, 2 * NV, zero_body, 0, unroll=UNROLL)

    def issue_prefetch(q):
        for rr in (0, 1):
            row = base + 2 * q + rr
            pltpu.async_copy(xp_hbm.at[row], KS[rr].at[pl.ds(0, N)], dma_sem)
            pltpu.async_copy(yp_hbm.at[row], KS[rr].at[pl.ds(N, M)], dma_sem)
            pltpu.async_copy(x_hbm.at[row], VS[rr].at[pl.ds(0, N)], dma_sem)
            pltpu.async_copy(y_hbm.at[row], VS[rr].at[pl.ds(N, M)], dma_sem)

    issue_prefetch(0)

    def pair_body(q, _):
        for rr in (0, 1):
            row = base + 2 * q + rr
            pltpu.make_async_copy(xp_hbm.at[row], KS[rr].at[pl.ds(0, N)], dma_sem).wait()
            pltpu.make_async_copy(yp_hbm.at[row], KS[rr].at[pl.ds(N, M)], dma_sem).wait()
            pltpu.make_async_copy(x_hbm.at[row], VS[rr].at[pl.ds(0, N)], dma_sem).wait()
            pltpu.make_async_copy(y_hbm.at[row], VS[rr].at[pl.ds(N, M)], dma_sem).wait()

        scale_vecs = [None]

        for p in range(NPASS):
            shift = 8 * p
            if p == 0:
                src_k, src_v = KS, VS
            elif p % 2 == 0:
                src_k, src_v = KB, VB
            else:
                src_k, src_v = KA, VA
            dst_k, dst_v = (KA, VA) if p % 2 == 0 else (KB, VB)

            def digit(rr, vreg_i, half, shift=shift, src_k=src_k):
                k = src_k[rr][pl.ds((vreg_i + half * NVH) * 16, 16)]
                ki = plsc.bitcast(k, jnp.int32)
                d = lax.shift_right_logical(ki, shift) & (NDIG - 1)
                return k, d

            if p == 0:
                def hist0_body(i, carry):
                    ax0, ay0, ax1, ay1 = carry
                    for rr in (0, 1):
                        _, da = digit(rr, i, 0)
                        plsc.addupdate_scatter(HI[rr], [(lanes << 8) | da], ones_i)
                        _, db = digit(rr, i, 1)
                        plsc.addupdate_scatter(HI[rr], [HB + ((lanes << 8) | db)], ones_i)
                    ax0 = ax0 + src_v[0][pl.ds(i * 16, 16)]
                    ay0 = ay0 + src_v[0][pl.ds((i + NVH) * 16, 16)]
                    ax1 = ax1 + src_v[1][pl.ds(i * 16, 16)]
                    ay1 = ay1 + src_v[1][pl.ds((i + NVH) * 16, 16)]
                    return ax0, ay0, ax1, ay1

                ax0, ay0, ax1, ay1 = lax.fori_loop(
                    0, NVH, hist0_body, (zeros_f, zeros_f, zeros_f, zeros_f),
                    unroll=UNROLL)
                scale_vecs[0] = (
                    (ones_f / (ones_f * jnp.sum(ax0)),
                     (-ones_f) / (ones_f * jnp.sum(ay0))),
                    (ones_f / (ones_f * jnp.sum(ax1)),
                     (-ones_f) / (ones_f * jnp.sum(ay1))),
                )
            else:
                def hist_body(i, _, digit=digit):
                    for rr in (0, 1):
                        _, da = digit(rr, i, 0)
                        plsc.addupdate_scatter(HI[rr], [(lanes << 8) | da], ones_i)
                        _, db = digit(rr, i, 1)
                        plsc.addupdate_scatter(HI[rr], [HB + ((lanes << 8) | db)], ones_i)
                    return 0

                lax.fori_loop(0, NVH, hist_body, 0, unroll=UNROLL)

            ngroups = (NDIG // 16) if p < NPASS - 1 else 4

            def off_group(j, carry):
                c0, c1 = carry
                outs = []
                for rr, c in ((0, c0), (1, c1)):
                    acc = zeros_i
                    es, has = [], []
                    for l in range(16):
                        sa = pl.ds(l * NDIG + j * 16, 16)
                        sb = pl.ds(HB + l * NDIG + j * 16, 16)
                        ha = HI[rr][sa]
                        hb = HI[rr][sb]
                        HI[rr][sa] = zeros_i
                        HI[rr][sb] = zeros_i
                        es.append(acc)
                        has.append(ha)
                        acc = acc + ha + hb
                    incl = plsc.cumsum(acc)
                    base_vec = c + incl - acc
                    for l in range(16):
                        oa = base_vec + es[l]
                        OF[rr][pl.ds(l * NDIG + j * 16, 16)] = oa
                        OF[rr][pl.ds(HB + l * NDIG + j * 16, 16)] = oa + has[l]
                    outs.append(c + incl[15])
                return tuple(outs)

            lax.fori_loop(0, ngroups, off_group, (jnp.int32(0), jnp.int32(0)))

            scales = scale_vecs[0] if p == 0 else None

            def perm_body(i, _, digit=digit, src_v=src_v, dst_k=dst_k,
                          dst_v=dst_v, scales=scales):
                for rr in (0, 1):
                    for half in (0, 1):
                        k, d = digit(rr, i, half)
                        v = src_v[rr][pl.ds((i + half * NVH) * 16, 16)]
                        taddr = half * HB + ((lanes << 8) | d)
                        g = plsc.load_gather(OF[rr], [taddr])
                        plsc.addupdate_scatter(OF[rr], [taddr], ones_i)
                        a = ((g & (NV - 1)) << 4) | lax.shift_right_logical(g, 8)
                        plsc.store_scatter(dst_k[rr], [a], k)
                        if scales is not None:
                            v = v * scales[rr][half]
                        plsc.store_scatter(dst_v[rr], [a], v)
                return 0

            lax.fori_loop(0, NVH, perm_body, 0, unroll=2)

            if p == 0:
                @pl.when(q + 1 < NPAIR)
                def _():
                    issue_prefetch(q + 1)

        def tot_body(i, carry):
            a0, a1 = carry
            return a0 + vb0[pl.ds(i * 16, 16)], a1 + vb1[pl.ds(i * 16, 16)]

        t0, t1 = lax.fori_loop(0, NV, tot_body, (zeros_f, zeros_f),
                               unroll=UNROLL)
        lb0 = plsc.cumsum(t0) - t0
        lb1 = plsc.cumsum(t1) - t1

        def loss_body(i, carry):
            k0, a0, s0, k1, a1, s1 = carry
            nk0 = kb0[pl.ds((i + 1) * 16, 16)]
            nk1 = kb1[pl.ds((i + 1) * 16, 16)]
            a0 = a0 + vb0[pl.ds(i * 16, 16)]
            a1 = a1 + vb1[pl.ds(i * 16, 16)]
            s0 = s0 + (nk0 - k0) * jnp.abs(lb0 + a0)
            s1 = s1 + (nk1 - k1) * jnp.abs(lb1 + a1)
            return nk0, a0, s0, nk1, a1, s1

        k0e, a0, s0, k1e, a1, s1 = lax.fori_loop(
            0, NV - 1, loss_body,
            (kb0[pl.ds(0, 16)], zeros_f, zeros_f,
             kb1[pl.ds(0, 16)], zeros_f, zeros_f), unroll=UNROLL)
        lane_p1 = jnp.minimum(lanes + 1, 15)
        for rr, (kb_r, vb_r, lb, ke, ac, ls) in enumerate(
                ((kb0, vb0, lb0, k0e, a0, s0), (kb1, vb1, lb1, k1e, a1, s1))):
            ac = ac + vb_r[pl.ds((NV - 1) * 16, 16)]
            d_end = lb + ac
            nxt = plsc.load_gather(kb_r, [lane_p1])
            seam = jnp.where(lanes < 15, (nxt - ke) * jnp.abs(d_end), 0.0)
            loss = jnp.sum(ls) + jnp.sum(seam)
            plsc.store_scatter(loss_buf, [ones_i * (2 * q + rr)],
                               zeros_f + loss, mask=lanes == 0)
        return 0

    lax.fori_loop(0, NPAIR, pair_body, 0)
    pltpu.sync_copy(loss_buf, out_hbm.at[pl.ds(base, RPW)])


@jax.jit
def kernel(x, y, x_pos, y_pos):
    mesh = plsc.VectorSubcoreMesh(core_axis_name="c", subcore_axis_name="s",
                                  num_cores=NC, num_subcores=NS)
    row_scratch = [
        pltpu.VMEM((W,), jnp.float32),
        pltpu.VMEM((W,), jnp.float32),
        pltpu.VMEM((W,), jnp.float32),
        pltpu.VMEM((W,), jnp.float32),
        pltpu.VMEM((W,), jnp.float32),
        pltpu.VMEM((W,), jnp.float32),
        pltpu.VMEM((2 * W,), jnp.int32),
        pltpu.VMEM((2 * W,), jnp.int32),
    ]
    f = pl.kernel(
        _sc_body,
        out_type=jax.ShapeDtypeStruct((B,), jnp.float32),
        mesh=mesh,
        scratch_types=row_scratch + row_scratch + [
            pltpu.VMEM((RPW,), jnp.float32),
            pltpu.SemaphoreType.DMA,
        ],
        compiler_params=pltpu.CompilerParams(needs_layout_passes=False),
    )
    return f(x, y, x_pos, y_pos)
